# baseline (speedup 1.0000x reference)
import jax
import jax.numpy as jnp
from jax import lax
from jax.experimental import pallas as pl
from jax.experimental.pallas import tpu as pltpu
from jax.experimental.pallas import tpu_sc as plsc

NC = 2
NS = 16
CH = 128
KC = 8
KL = 8
SL = 6
HALF = 32
LB = 1024
NLAYER = 3

_MESH = plsc.VectorSubcoreMesh(
    core_axis_name="c", subcore_axis_name="s", num_cores=NC, num_subcores=NS
)
_SC_PARAMS = pltpu.CompilerParams(use_tc_tiling_on_sc=False)


def _hist_sc(e_all, nr, cpt):
    stripe = nr // NS
    qrt = stripe // 8

    @pl.kernel(
        out_type=jax.ShapeDtypeStruct((NC * nr, HALF), jnp.float32),
        mesh=_MESH,
        compiler_params=_SC_PARAMS,
        scratch_types=[
            pltpu.VMEM_SHARED((nr, HALF), jnp.float32),
            pltpu.VMEM((KC, CH), jnp.int32),
            pltpu.VMEM((CH, HALF), jnp.float32),
            pltpu.VMEM((qrt, HALF), jnp.float32),
            pltpu.SemaphoreType.DMA,
        ],
    )
    def hist(e_ref, deg_ref, acc, ibuf, ones_v, stage, sem):
        c = lax.axis_index("c")
        s = lax.axis_index("s")

        @pl.loop(0, qrt)
        def _(i):
            @pl.loop(0, HALF, step=16)
            def _(q):
                stage[i, pl.ds(q, 16)] = jnp.zeros((16,), jnp.float32)

        @pl.loop(0, CH)
        def _(i):
            @pl.loop(0, HALF, step=16)
            def _(q):
                ones_v[i, pl.ds(q, 16)] = jnp.ones((16,), jnp.float32)

        @pl.loop(0, 8)
        def _(q):
            pltpu.sync_copy(stage, acc.at[pl.ds(s * stripe + q * qrt, qrt)])

        plsc.subcore_barrier()

        @pl.loop(0, cpt // KC)
        def _(j):
            pltpu.sync_copy(e_ref.at[c, s, pl.ds(j * KC, KC)], ibuf)
            descs = [
                pltpu.async_copy(ones_v, acc.at[ibuf.at[k]], sem, add=True)
                for k in range(KC)
            ]
            for d in descs:
                d.wait()

        plsc.subcore_barrier()

        @pl.loop(0, 8)
        def _(q):
            pltpu.sync_copy(acc.at[pl.ds(s * stripe + q * qrt, qrt)], stage)
            pltpu.sync_copy(
                stage, deg_ref.at[pl.ds(c * nr + s * stripe + q * qrt, qrt)]
            )

    return hist(e_all)


def _mega_sc(src2, dstp, g0, co, nr, cpt):
    stripe = nr // NS
    qrt = stripe // 64

    @pl.kernel(
        out_type=tuple(
            jax.ShapeDtypeStruct((NC * nr, HALF), jnp.float32)
            for _ in range(NLAYER)
        ),
        mesh=_MESH,
        compiler_params=_SC_PARAMS,
        scratch_types=[
            pltpu.VMEM_SHARED((nr, HALF), jnp.float32),
            pltpu.HBM((NC * nr, HALF), jnp.float32),
            pltpu.VMEM((SL, CH, HALF), jnp.float32),
            pltpu.VMEM((KL, CH), jnp.int32),
            pltpu.VMEM((KL, CH), jnp.int32),
            pltpu.VMEM((qrt, HALF), jnp.float32),
            pltpu.VMEM((qrt, HALF), jnp.float32),
            pltpu.SemaphoreType.DMA,
            pltpu.SemaphoreType.DMA,
        ],
    )
    def mega(
        s_ref, d_ref, g_ref, co_ref,
        a0_ref, a1_ref, a2_ref,
        acc, gscr, sbuf, isrc, idst, stage, costage, sem, sem2,
    ):
        c = lax.axis_index("c")
        s = lax.axis_index("s")
        aggs = (a0_ref, a1_ref, a2_ref)

        @pl.loop(0, qrt)
        def _(i):
            @pl.loop(0, HALF, step=16)
            def _(q):
                stage[i, pl.ds(q, 16)] = jnp.zeros((16,), jnp.float32)

        @pl.loop(0, 64)
        def _(q):
            pltpu.sync_copy(stage, acc.at[pl.ds(s * stripe + q * qrt, qrt)])

        plsc.subcore_barrier()

        for l in range(NLAYER):
            src_g = g_ref if l == 0 else gscr
            a_ref = aggs[l]

            @pl.loop(0, cpt // KL)
            def _(j):
                pltpu.sync_copy(s_ref.at[c, s, pl.ds(j * KL, KL)], isrc)
                pltpu.sync_copy(d_ref.at[s, pl.ds(j * KL, KL)], idst)
                descs = [
                    pltpu.async_copy(src_g.at[isrc.at[t]], sbuf.at[t], sem)
                    for t in range(SL)
                ]
                for k in range(KL):
                    descs[k % SL].wait()
                    pltpu.sync_copy(
                        sbuf.at[k % SL], acc.at[idst.at[k]], add=True
                    )
                    if k + SL < KL:
                        descs[k % SL] = pltpu.async_copy(
                            src_g.at[isrc.at[k + SL]], sbuf.at[k % SL], sem
                        )

            plsc.subcore_barrier()

            @pl.loop(0, 64)
            def _(q):
                loc = pl.ds(s * stripe + q * qrt, qrt)
                glo = pl.ds(c * nr + s * stripe + q * qrt, qrt)
                if l < NLAYER - 1:
                    dco = pltpu.async_copy(co_ref.at[glo], costage, sem2)
                pltpu.sync_copy(acc.at[loc], stage)
                pltpu.sync_copy(stage, a_ref.at[glo])
                if l < NLAYER - 1:
                    dco.wait()

                    @pl.loop(0, qrt)
                    def _(i):
                        @pl.loop(0, HALF, step=16)
                        def _(p):
                            stage[i, pl.ds(p, 16)] = (
                                stage[i, pl.ds(p, 16)]
                                * costage[i, pl.ds(p, 16)]
                            )

                    pltpu.sync_copy(stage, gscr.at[glo])

                    @pl.loop(0, qrt)
                    def _(i):
                        @pl.loop(0, HALF, step=16)
                        def _(p):
                            stage[i, pl.ds(p, 16)] = jnp.zeros(
                                (16,), jnp.float32
                            )

                    pltpu.sync_copy(stage, acc.at[loc])

            plsc.subcore_barrier()

    return mega(src2, dstp, g0, co)


def _ew_call(body, out_shape, *args):
    rr = args[0].shape[1]
    rb = LB * HALF // 128
    specs = [
        pl.BlockSpec((NC, rb, 128), lambda b: (0, b, 0)) for _ in args
    ]
    return pl.pallas_call(
        body,
        grid=(rr // rb,),
        in_specs=specs,
        out_specs=pl.BlockSpec(
            (NC, rb, 128) if len(out_shape) == 3 else (rb, 128),
            (lambda b: (0, b, 0)) if len(out_shape) == 3 else (lambda b: (b, 0)),
        ),
        out_shape=jax.ShapeDtypeStruct(out_shape, jnp.float32),
    )(*args)


def _prep_tc(h0h, degx, rr):
    def body(h_ref, d_ref, g_ref, co_ref):
        csrc = lax.rsqrt(jnp.maximum(d_ref[0], 1.0))
        cdst = lax.rsqrt(jnp.maximum(d_ref[1], 1.0))
        g_ref[...] = h_ref[...] * csrc[None]
        co_ref[...] = jnp.broadcast_to(
            (csrc * cdst)[None], co_ref.shape
        )

    rb = LB * HALF // 128
    return pl.pallas_call(
        body,
        grid=(rr // rb,),
        in_specs=[
            pl.BlockSpec((NC, rb, 128), lambda b: (0, b, 0)),
            pl.BlockSpec((NC, rb, 128), lambda b: (0, b, 0)),
        ],
        out_specs=[
            pl.BlockSpec((NC, rb, 128), lambda b: (0, b, 0)),
            pl.BlockSpec((NC, rb, 128), lambda b: (0, b, 0)),
        ],
        out_shape=[
            jax.ShapeDtypeStruct((NC, rr, 128), jnp.float32),
            jax.ShapeDtypeStruct((NC, rr, 128), jnp.float32),
        ],
    )(h0h, degx)


def _final_tc(h0h, a0, a1, a2, degx, rr):
    def body(h_ref, a0_ref, a1_ref, a2_ref, d_ref, o_ref):
        cdst = lax.rsqrt(jnp.maximum(d_ref[1], 1.0))
        ssum = a0_ref[...] + a1_ref[...] + a2_ref[...]
        o_ref[...] = (h_ref[...] + ssum * cdst[None]) * 0.25

    return _ew_call(body, (NC, rr, 128), h0h, a0, a1, a2, degx)


def kernel(edge_index, user_emb, item_emb):
    nu, d_full = user_emb.shape
    ni = item_emb.shape[0]
    n = nu + ni
    e = edge_index.shape[1]

    nr = ((n + NS + LB - 1) // LB) * LB
    ndump = nr - n
    gran = KC * KL // 2
    cpt = ((e + NS * CH - 1) // (NS * CH) + gran - 1) // gran * gran
    epad = NS * cpt * CH
    rr = nr * HALF // 128

    src = edge_index[0]
    dst = edge_index[1]
    pad_idx = n + (jnp.arange(epad - e, dtype=jnp.int32) % ndump)
    srcp = jnp.concatenate([src, pad_idx]).reshape(NS, cpt, CH)
    dstp = jnp.concatenate([dst, pad_idx]).reshape(NS, cpt, CH)
    src2 = jnp.stack([srcp, srcp + nr])
    e_all = jnp.stack([srcp, dstp])

    degx = _hist_sc(e_all, nr, cpt).reshape(NC, rr, 128)

    h0 = jnp.concatenate([user_emb, item_emb], axis=0)
    h0p = jnp.pad(h0, ((0, nr - n), (0, 0)))
    h0h = jnp.stack([h0p[:, :HALF], h0p[:, HALF:]]).reshape(NC, rr, 128)

    g0, co = _prep_tc(h0h, degx, rr)
    aggs = _mega_sc(
        src2, dstp, g0.reshape(NC * nr, HALF), co.reshape(NC * nr, HALF),
        nr, cpt,
    )
    a0, a1, a2 = (a.reshape(NC, rr, 128) for a in aggs)

    outh = _final_tc(h0h, a0, a1, a2, degx, rr)
    outf = jnp.transpose(outh.reshape(NC, nr, HALF), (1, 0, 2)).reshape(nr, d_full)
    return outf[:nu], outf[nu:n]


# speedup vs baseline: 1.2167x; 1.2167x over previous
import jax
import jax.numpy as jnp
from jax import lax
from jax.experimental import pallas as pl
from jax.experimental.pallas import tpu as pltpu
from jax.experimental.pallas import tpu_sc as plsc

NC = 2
NS = 16
CH = 128
KC = 8
KL = 14
SL = 6
HALF = 32
LB = 1024
NLAYER = 3

_MESH = plsc.VectorSubcoreMesh(
    core_axis_name="c", subcore_axis_name="s", num_cores=NC, num_subcores=NS
)
_SC_PARAMS = pltpu.CompilerParams(use_tc_tiling_on_sc=False)


def _hist_sc(e_all, nr, cpt):
    stripe = nr // NS
    qrt = stripe // 8

    @pl.kernel(
        out_type=jax.ShapeDtypeStruct((NC * nr, HALF), jnp.float32),
        mesh=_MESH,
        compiler_params=_SC_PARAMS,
        scratch_types=[
            pltpu.VMEM_SHARED((nr, HALF), jnp.float32),
            pltpu.VMEM((KC, CH), jnp.int32),
            pltpu.VMEM((CH, HALF), jnp.float32),
            pltpu.VMEM((qrt, HALF), jnp.float32),
            pltpu.SemaphoreType.DMA,
        ],
    )
    def hist(e_ref, deg_ref, acc, ibuf, ones_v, stage, sem):
        c = lax.axis_index("c")
        s = lax.axis_index("s")

        @pl.loop(0, qrt)
        def _(i):
            @pl.loop(0, HALF, step=16)
            def _(q):
                stage[i, pl.ds(q, 16)] = jnp.zeros((16,), jnp.float32)

        @pl.loop(0, CH)
        def _(i):
            @pl.loop(0, HALF, step=16)
            def _(q):
                ones_v[i, pl.ds(q, 16)] = jnp.ones((16,), jnp.float32)

        @pl.loop(0, 8)
        def _(q):
            pltpu.sync_copy(stage, acc.at[pl.ds(s * stripe + q * qrt, qrt)])

        plsc.subcore_barrier()

        @pl.loop(0, cpt // KC)
        def _(j):
            pltpu.sync_copy(e_ref.at[c, s, pl.ds(j * KC, KC)], ibuf)
            descs = [
                pltpu.async_copy(ones_v, acc.at[ibuf.at[k]], sem, add=True)
                for k in range(KC)
            ]
            for d in descs:
                d.wait()

        plsc.subcore_barrier()

        @pl.loop(0, 8)
        def _(q):
            pltpu.sync_copy(acc.at[pl.ds(s * stripe + q * qrt, qrt)], stage)
            pltpu.sync_copy(
                stage, deg_ref.at[pl.ds(c * nr + s * stripe + q * qrt, qrt)]
            )

    return hist(e_all)


def _layer_sc(src2, dstp, g2, nr, cpt):
    stripe = nr // NS
    qrt = stripe // 64

    @pl.kernel(
        out_type=jax.ShapeDtypeStruct((NC * nr, HALF), jnp.float32),
        mesh=_MESH,
        compiler_params=_SC_PARAMS,
        scratch_types=[
            pltpu.VMEM_SHARED((nr, HALF), jnp.float32),
            pltpu.VMEM((SL, CH, HALF), jnp.float32),
            pltpu.VMEM((KL, CH), jnp.int32),
            pltpu.VMEM((KL, CH), jnp.int32),
            pltpu.VMEM((qrt, HALF), jnp.float32),
            pltpu.SemaphoreType.DMA,
        ],
    )
    def layer(s_ref, d_ref, g_ref, agg_ref, acc, sbuf, isrc, idst, stage, sem):
        c = lax.axis_index("c")
        s = lax.axis_index("s")

        @pl.loop(0, qrt)
        def _(i):
            @pl.loop(0, HALF, step=16)
            def _(q):
                stage[i, pl.ds(q, 16)] = jnp.zeros((16,), jnp.float32)

        @pl.loop(0, 64)
        def _(q):
            pltpu.sync_copy(stage, acc.at[pl.ds(s * stripe + q * qrt, qrt)])

        plsc.subcore_barrier()

        @pl.loop(0, cpt // KL)
        def _(j):
            pltpu.sync_copy(s_ref.at[c, s, pl.ds(j * KL, KL)], isrc)
            pltpu.sync_copy(d_ref.at[s, pl.ds(j * KL, KL)], idst)
            descs = [
                pltpu.async_copy(g_ref.at[isrc.at[t]], sbuf.at[t], sem)
                for t in range(SL)
            ]
            for k in range(KL):
                descs[k % SL].wait()
                pltpu.sync_copy(
                    sbuf.at[k % SL], acc.at[idst.at[k]], add=True
                )
                if k + SL < KL:
                    descs[k % SL] = pltpu.async_copy(
                        g_ref.at[isrc.at[k + SL]], sbuf.at[k % SL], sem
                    )

        plsc.subcore_barrier()

        @pl.loop(0, 64)
        def _(q):
            pltpu.sync_copy(acc.at[pl.ds(s * stripe + q * qrt, qrt)], stage)
            pltpu.sync_copy(
                stage, agg_ref.at[pl.ds(c * nr + s * stripe + q * qrt, qrt)]
            )

    return layer(src2, dstp, g2)


def _ew_call(body, out_shape, *args):
    rr = args[0].shape[1]
    rb = LB * HALF // 128
    specs = [
        pl.BlockSpec((NC, rb, 128), lambda b: (0, b, 0)) for _ in args
    ]
    return pl.pallas_call(
        body,
        grid=(rr // rb,),
        in_specs=specs,
        out_specs=pl.BlockSpec(
            (NC, rb, 128) if len(out_shape) == 3 else (rb, 128),
            (lambda b: (0, b, 0)) if len(out_shape) == 3 else (lambda b: (b, 0)),
        ),
        out_shape=jax.ShapeDtypeStruct(out_shape, jnp.float32),
    )(*args)


def _prep_tc(h0h, degx, rr):
    def body(h_ref, d_ref, o_ref):
        csrc = lax.rsqrt(jnp.maximum(d_ref[0], 1.0))
        o_ref[...] = h_ref[...] * csrc[None]

    return _ew_call(body, (NC, rr, 128), h0h, degx)


def _scale_tc(agg, degx, rr):
    def body(a_ref, d_ref, o_ref):
        co = lax.rsqrt(jnp.maximum(d_ref[0], 1.0)) * lax.rsqrt(
            jnp.maximum(d_ref[1], 1.0)
        )
        o_ref[...] = a_ref[...] * co[None]

    return _ew_call(body, (NC, rr, 128), agg, degx)


def _final_tc(h0h, a0, a1, a2, degx, rr):
    def body(h_ref, a0_ref, a1_ref, a2_ref, d_ref, o_ref):
        cdst = lax.rsqrt(jnp.maximum(d_ref[1], 1.0))
        ssum = a0_ref[...] + a1_ref[...] + a2_ref[...]
        o_ref[...] = (h_ref[...] + ssum * cdst[None]) * 0.25

    return _ew_call(body, (NC, rr, 128), h0h, a0, a1, a2, degx)


def kernel(edge_index, user_emb, item_emb):
    nu, d_full = user_emb.shape
    ni = item_emb.shape[0]
    n = nu + ni
    e = edge_index.shape[1]

    nr = ((n + NS + LB - 1) // LB) * LB
    ndump = nr - n
    gran = KC * KL // 2
    cpt = ((e + NS * CH - 1) // (NS * CH) + gran - 1) // gran * gran
    epad = NS * cpt * CH
    rr = nr * HALF // 128

    src = edge_index[0]
    dst = edge_index[1]
    pad_idx = n + (jnp.arange(epad - e, dtype=jnp.int32) % ndump)
    srcp = jnp.concatenate([src, pad_idx]).reshape(NS, cpt, CH)
    dstp = jnp.concatenate([dst, pad_idx]).reshape(NS, cpt, CH)
    src2 = jnp.stack([srcp, srcp + nr])
    e_all = jnp.stack([srcp, dstp])

    degx = _hist_sc(e_all, nr, cpt).reshape(NC, rr, 128)

    h0 = jnp.concatenate([user_emb, item_emb], axis=0)
    h0p = jnp.pad(h0, ((0, nr - n), (0, 0)))
    h0h = jnp.stack([h0p[:, :HALF], h0p[:, HALF:]]).reshape(NC, rr, 128)

    g = _prep_tc(h0h, degx, rr)
    aggs = []
    for layer_i in range(NLAYER):
        agg = _layer_sc(src2, dstp, g.reshape(NC * nr, HALF), nr, cpt)
        agg = agg.reshape(NC, rr, 128)
        aggs.append(agg)
        if layer_i < NLAYER - 1:
            g = _scale_tc(agg, degx, rr)

    outh = _final_tc(h0h, aggs[0], aggs[1], aggs[2], degx, rr)
    outf = jnp.transpose(outh.reshape(NC, nr, HALF), (1, 0, 2)).reshape(nr, d_full)
    return outf[:nu], outf[nu:n]
